# baseline (speedup 1.0000x reference)
import functools

import jax
import jax.numpy as jnp
from jax import lax
from jax.experimental import pallas as pl
from jax.experimental.pallas import tpu as pltpu
from jax.experimental.pallas import tpu_sc as plsc

N = 10000
F = 128
NC, NS = 2, 16
NW = NC * NS
CHUNK = 128
RPT = 640
N_PAD = NS * RPT
DW = 128
DEG_Q = 8

_MESH = plsc.VectorSubcoreMesh(core_axis_name="c", subcore_axis_name="s")


def _sc_degree(edges, zeros_col, ones_col, nchunk):
    q = min(DEG_Q, nchunk)

    @functools.partial(
        pl.kernel,
        out_type=jax.ShapeDtypeStruct((NC, N_PAD, DW), jnp.float32),
        mesh=_MESH,
        scratch_types=[
            pltpu.VMEM((nchunk, CHUNK), jnp.int32),
            pltpu.VMEM((CHUNK, DW), jnp.float32),
            pltpu.VMEM_SHARED((N_PAD, DW), jnp.float32),
            pltpu.SemaphoreType.DMA,
        ],
    )
    def k(e_hbm, z_hbm, o_hbm, out_hbm, dst2d, ones_v, acc, sem):
        c = lax.axis_index("c")
        s = lax.axis_index("s")
        wid = c * NS + s
        pltpu.sync_copy(z_hbm, acc.at[pl.ds(s * RPT, RPT)])
        pltpu.sync_copy(o_hbm, ones_v)
        pltpu.sync_copy(e_hbm.at[1, wid], dst2d)
        plsc.subcore_barrier()

        def wait_one():
            pltpu.make_async_copy(ones_v, acc.at[dst2d.at[0]], sem).wait()

        def issue(j, carry):
            pltpu.async_copy(ones_v, acc.at[dst2d.at[j]], sem, add=True)

            @pl.when(j >= q)
            def _():
                wait_one()
            return carry

        lax.fori_loop(0, nchunk, issue, 0)
        for _ in range(q):
            wait_one()
        plsc.subcore_barrier()
        pltpu.sync_copy(acc.at[pl.ds(s * RPT, RPT)],
                        out_hbm.at[c, pl.ds(s * RPT, RPT)])

    return k(edges, zeros_col, ones_col)


def _sc_aggregate(y, edges, zeros_blk, nchunk):
    assert nchunk % 4 == 0 and nchunk >= 8

    @functools.partial(
        pl.kernel,
        out_type=jax.ShapeDtypeStruct((NC, N_PAD, F), jnp.float32),
        mesh=_MESH,
        scratch_types=[
            pltpu.VMEM((nchunk, CHUNK), jnp.int32),
            pltpu.VMEM((4, CHUNK), jnp.int32),
            pltpu.VMEM((2, CHUNK, F), jnp.float32),
            pltpu.VMEM_SHARED((N_PAD, F), jnp.float32),
            [pltpu.SemaphoreType.DMA] * 4,
            [pltpu.SemaphoreType.DMA] * 2,
            [pltpu.SemaphoreType.DMA] * 2,
        ],
    )
    def k(y_hbm, e_hbm, z_hbm, out_hbm, dst2d, srcr, rows, acc,
          isems, gsems, ssems):
        c = lax.axis_index("c")
        s = lax.axis_index("s")
        wid = c * NS + s
        pltpu.sync_copy(z_hbm, acc.at[pl.ds(s * RPT, RPT)])
        pltpu.sync_copy(e_hbm.at[1, wid], dst2d)
        plsc.subcore_barrier()

        def load_src(j, sl):
            pltpu.async_copy(e_hbm.at[0, wid, j], srcr.at[sl], isems[sl])

        def wait_src(sl):
            pltpu.make_async_copy(e_hbm.at[0, wid, 0], srcr.at[sl],
                                  isems[sl]).wait()

        def gather(sl, rb):
            pltpu.async_copy(y_hbm.at[srcr.at[sl]], rows.at[rb], gsems[rb])

        def wait_gather(rb):
            pltpu.make_async_copy(y_hbm.at[srcr.at[0]], rows.at[rb],
                                  gsems[rb]).wait()

        def scatter(j, rb):
            pltpu.async_copy(rows.at[rb], acc.at[dst2d.at[j]], ssems[rb],
                             add=True)

        def wait_scatter(rb):
            pltpu.make_async_copy(rows.at[0], acc.at[dst2d.at[0]],
                                  ssems[rb]).wait()

        pltpu.sync_copy(e_hbm.at[0, wid, 0], srcr.at[0])
        load_src(1, 1)
        gather(0, 0)

        def body(g, carry):
            for b in range(4):
                j = g * 4 + b
                s1 = (b + 1) % 4
                r1 = (b + 1) % 2

                @pl.when(j + 2 < nchunk)
                def _():
                    load_src(j + 2, (b + 2) % 4)

                @pl.when(jnp.logical_and(j >= 1, j + 1 < nchunk))
                def _():
                    wait_scatter(r1)

                @pl.when(j + 1 < nchunk)
                def _():
                    wait_src(s1)
                    gather(s1, r1)

                wait_gather(b % 2)
                scatter(j, b % 2)
            return carry

        lax.fori_loop(0, nchunk // 4, body, 0)

        wait_scatter(0)
        wait_scatter(1)
        plsc.subcore_barrier()
        pltpu.sync_copy(acc.at[pl.ds(s * RPT, RPT)],
                        out_hbm.at[c, pl.ds(s * RPT, RPT)])

    return k(y, edges, zeros_blk)


_BR = 2000
_GRID = (N + _BR - 1) // _BR


def _tc_first(x, W, degp):
    def body(x_ref, w_ref, d0_ref, d1_ref, y_ref, dis_ref):
        deg = d0_ref[0][:, 0:1] + d1_ref[0][:, 0:1] + 1.0
        dis = lax.rsqrt(deg)
        xw = lax.dot_general(x_ref[...], w_ref[...],
                             (((1,), (1,)), ((), ())),
                             preferred_element_type=jnp.float32)
        y_ref[...] = xw * dis
        dis_ref[...] = dis

    return pl.pallas_call(
        body,
        grid=(_GRID,),
        in_specs=[
            pl.BlockSpec((_BR, F), lambda i: (i, 0)),
            pl.BlockSpec((F, F), lambda i: (0, 0)),
            pl.BlockSpec((1, _BR, DW), lambda i: (0, i, 0)),
            pl.BlockSpec((1, _BR, DW), lambda i: (1, i, 0)),
        ],
        out_specs=[
            pl.BlockSpec((_BR, F), lambda i: (i, 0)),
            pl.BlockSpec((_BR, 1), lambda i: (i, 0)),
        ],
        out_shape=[
            jax.ShapeDtypeStruct((N, F), jnp.float32),
            jax.ShapeDtypeStruct((N, 1), jnp.float32),
        ],
    )(x, W, degp, degp)


def _tc_mid(parts, y1, dis, b1, W2):
    def body(p0_ref, p1_ref, y1_ref, dis_ref, b_ref, w_ref, y2_ref):
        dis = dis_ref[...]
        h = (p0_ref[0] + p1_ref[0] + y1_ref[...]) * dis + b_ref[...]
        hw = lax.dot_general(h, w_ref[...], (((1,), (1,)), ((), ())),
                             preferred_element_type=jnp.float32)
        y2_ref[...] = hw * dis

    return pl.pallas_call(
        body,
        grid=(_GRID,),
        in_specs=[
            pl.BlockSpec((1, _BR, F), lambda i: (0, i, 0)),
            pl.BlockSpec((1, _BR, F), lambda i: (1, i, 0)),
            pl.BlockSpec((_BR, F), lambda i: (i, 0)),
            pl.BlockSpec((_BR, 1), lambda i: (i, 0)),
            pl.BlockSpec((1, F), lambda i: (0, 0)),
            pl.BlockSpec((F, F), lambda i: (0, 0)),
        ],
        out_specs=pl.BlockSpec((_BR, F), lambda i: (i, 0)),
        out_shape=jax.ShapeDtypeStruct((N, F), jnp.float32),
    )(parts, parts, y1, dis, b1, W2)


def _tc_last(parts, y2, dis, b2):
    def body(p0_ref, p1_ref, y2_ref, dis_ref, b_ref, out_ref):
        out_ref[...] = ((p0_ref[0] + p1_ref[0] + y2_ref[...])
                        * dis_ref[...] + b_ref[...])

    return pl.pallas_call(
        body,
        grid=(_GRID,),
        in_specs=[
            pl.BlockSpec((1, _BR, F), lambda i: (0, i, 0)),
            pl.BlockSpec((1, _BR, F), lambda i: (1, i, 0)),
            pl.BlockSpec((_BR, F), lambda i: (i, 0)),
            pl.BlockSpec((_BR, 1), lambda i: (i, 0)),
            pl.BlockSpec((1, F), lambda i: (0, 0)),
        ],
        out_specs=pl.BlockSpec((_BR, F), lambda i: (i, 0)),
        out_shape=jax.ShapeDtypeStruct((N, F), jnp.float32),
    )(parts, parts, y2, dis, b2)


def kernel(x, edge_index, W1, b1, W2, b2):
    E = edge_index.shape[1]
    group = CHUNK * 4
    ept = ((E + NW - 1) // NW + group - 1) // group * group
    nchunk = ept // CHUNK
    e_pad = ept * NW
    e = edge_index.astype(jnp.int32)
    pad = e_pad - E
    src = jnp.concatenate([e[0], jnp.zeros((pad,), jnp.int32)])
    dst = jnp.concatenate([e[1], jnp.full((pad,), N, jnp.int32)])
    edges = jnp.stack([src, dst]).reshape(2, NW, nchunk, CHUNK)

    zeros_blk = jnp.zeros((RPT, F), jnp.float32)
    zeros_col = jnp.zeros((RPT, DW), jnp.float32)
    ones_col = jnp.ones((CHUNK, DW), jnp.float32)
    b1r = b1.reshape(1, F)
    b2r = b2.reshape(1, F)

    degp = _sc_degree(edges, zeros_col, ones_col, nchunk)
    y1, dis = _tc_first(x, W1, degp)
    s1 = _sc_aggregate(y1, edges, zeros_blk, nchunk)
    y2 = _tc_mid(s1, y1, dis, b1r, W2)
    s2 = _sc_aggregate(y2, edges, zeros_blk, nchunk)
    return _tc_last(s2, y2, dis, b2r)


# speedup vs baseline: 1.0250x; 1.0250x over previous
import functools

import jax
import jax.numpy as jnp
from jax import lax
from jax.experimental import pallas as pl
from jax.experimental.pallas import tpu as pltpu
from jax.experimental.pallas import tpu_sc as plsc

N = 10000
F = 128
NC, NS = 2, 16
NW = NC * NS
CHUNK = 128
RPT = 632
N_PAD = NS * RPT
DW = 128
DEG_Q = 8
GRP = 8
C0_CNT = 128
C1_CNT = 32
DEG_CNT = (C0_CNT + C1_CNT) // 2

_MESH = plsc.VectorSubcoreMesh(core_axis_name="c", subcore_axis_name="s")


def _al8(i):
    return pl.multiple_of(i, 8)


def _sc_degree(edges, zeros_col, ones_col):
    @functools.partial(
        pl.kernel,
        out_type=jax.ShapeDtypeStruct((NC, N_PAD, DW), jnp.float32),
        mesh=_MESH,
        scratch_types=[
            pltpu.VMEM((DEG_CNT, CHUNK), jnp.int32),
            pltpu.VMEM((CHUNK, DW), jnp.float32),
            pltpu.VMEM_SHARED((N_PAD, DW), jnp.float32),
            pltpu.SemaphoreType.DMA,
        ],
    )
    def k(e_hbm, z_hbm, o_hbm, out_hbm, dst2d, ones_v, acc, sem):
        c = lax.axis_index("c")
        s = lax.axis_index("s")
        wid = c * NS + s
        row0 = _al8(s * RPT)
        pltpu.sync_copy(z_hbm, acc.at[pl.ds(row0, RPT)])
        pltpu.sync_copy(o_hbm, ones_v)
        pltpu.sync_copy(e_hbm.at[1, pl.ds(_al8(wid * DEG_CNT), DEG_CNT)],
                        dst2d)
        plsc.subcore_barrier()

        def wait_one():
            pltpu.make_async_copy(ones_v, acc.at[dst2d.at[0]], sem).wait()

        def issue(j, carry):
            pltpu.async_copy(ones_v, acc.at[dst2d.at[j]], sem, add=True)

            @pl.when(j >= DEG_Q)
            def _():
                wait_one()
            return carry

        lax.fori_loop(0, DEG_CNT, issue, 0)
        for _ in range(DEG_Q):
            wait_one()
        plsc.subcore_barrier()
        pltpu.sync_copy(acc.at[pl.ds(row0, RPT)],
                        out_hbm.at[c, pl.ds(row0, RPT)])

    return k(edges, zeros_col, ones_col)


def _sc_aggregate(y, edges, zeros_blk):
    @functools.partial(
        pl.kernel,
        out_type=jax.ShapeDtypeStruct((NC, N_PAD, F), jnp.float32),
        mesh=_MESH,
        scratch_types=[
            pltpu.VMEM((2, GRP, CHUNK), jnp.int32),
            pltpu.VMEM((2, GRP, CHUNK), jnp.int32),
            pltpu.VMEM((2, CHUNK, F), jnp.float32),
            pltpu.VMEM_SHARED((N_PAD, F), jnp.float32),
            [pltpu.SemaphoreType.DMA] * 2,
            [pltpu.SemaphoreType.DMA] * 2,
            [pltpu.SemaphoreType.DMA] * 2,
            [pltpu.SemaphoreType.DMA] * 2,
        ],
    )
    def k(y_hbm, e_hbm, z_hbm, out_hbm, srcg, dstg, rows, acc,
          isems, dsems, gsems, ssems):
        c = lax.axis_index("c")
        s = lax.axis_index("s")
        cnt = jnp.where(c == 0, C0_CNT, C1_CNT)
        trip = jnp.where(c == 0, C0_CNT // (2 * GRP), C1_CNT // (2 * GRP))
        base = jnp.where(c == 0, s * C0_CNT, NS * C0_CNT + s * C1_CNT)
        row0 = _al8(s * RPT)
        pltpu.sync_copy(z_hbm, acc.at[pl.ds(row0, RPT)])

        def load_grp(gg, sl):
            at = pl.ds(_al8(base + gg * GRP), GRP)
            pltpu.async_copy(e_hbm.at[0, at], srcg.at[sl], isems[sl])
            pltpu.async_copy(e_hbm.at[1, at], dstg.at[sl], dsems[sl])

        def wait_grp(sl):
            at = pl.ds(0, GRP)
            pltpu.make_async_copy(e_hbm.at[0, at], srcg.at[sl],
                                  isems[sl]).wait()
            pltpu.make_async_copy(e_hbm.at[1, at], dstg.at[sl],
                                  dsems[sl]).wait()

        def gather(sl, kk, rb):
            pltpu.async_copy(y_hbm.at[srcg.at[sl, kk]], rows.at[rb],
                             gsems[rb])

        def wait_gather(rb):
            pltpu.make_async_copy(y_hbm.at[srcg.at[0, 0]], rows.at[rb],
                                  gsems[rb]).wait()

        def scatter(sl, kk, rb):
            pltpu.async_copy(rows.at[rb], acc.at[dstg.at[sl, kk]],
                             ssems[rb], add=True)

        def wait_scatter(rb):
            pltpu.make_async_copy(rows.at[0], acc.at[dstg.at[0, 0]],
                                  ssems[rb]).wait()

        load_grp(0, 0)
        wait_grp(0)
        plsc.subcore_barrier()
        gather(0, 0, 0)

        def body(t, carry):
            for g2 in range(2):
                sg, other = g2, 1 - g2
                gbase = t * 2 * GRP + g2 * GRP
                gg = t * 2 + g2
                for kk in range(GRP):
                    j = gbase + kk
                    rb = kk % 2
                    rb1 = (kk + 1) % 2

                    if kk == 0:
                        @pl.when(jnp.logical_and(j >= 1, j + 1 < cnt))
                        def _():
                            wait_scatter(1)
                        @pl.when(j + GRP < cnt)
                        def _():
                            load_grp(gg + 1, other)
                    else:
                        @pl.when(jnp.logical_and(j >= 1, j + 1 < cnt))
                        def _():
                            wait_scatter(rb1)

                    if kk == GRP - 1:
                        @pl.when(j + 1 < cnt)
                        def _():
                            wait_grp(other)
                            gather(other, 0, rb1)
                    else:
                        @pl.when(j + 1 < cnt)
                        def _():
                            gather(sg, kk + 1, rb1)

                    wait_gather(rb)
                    scatter(sg, kk, rb)
            return carry

        lax.fori_loop(0, trip, body, 0)

        wait_scatter(0)
        wait_scatter(1)
        plsc.subcore_barrier()
        pltpu.sync_copy(acc.at[pl.ds(row0, RPT)],
                        out_hbm.at[c, pl.ds(row0, RPT)])

    return k(y, edges, zeros_blk)


_BR = 2000
_GRID = (N + _BR - 1) // _BR


def _tc_first(x, W, degp):
    def body(x_ref, w_ref, d0_ref, d1_ref, y_ref, dis_ref):
        deg = d0_ref[0][:, 0:1] + d1_ref[0][:, 0:1] + 1.0
        dis = lax.rsqrt(deg)
        xw = lax.dot_general(x_ref[...], w_ref[...],
                             (((1,), (1,)), ((), ())),
                             preferred_element_type=jnp.float32)
        y_ref[...] = xw * dis
        dis_ref[...] = dis

    return pl.pallas_call(
        body,
        grid=(_GRID,),
        in_specs=[
            pl.BlockSpec((_BR, F), lambda i: (i, 0)),
            pl.BlockSpec((F, F), lambda i: (0, 0)),
            pl.BlockSpec((1, _BR, DW), lambda i: (0, i, 0)),
            pl.BlockSpec((1, _BR, DW), lambda i: (1, i, 0)),
        ],
        out_specs=[
            pl.BlockSpec((_BR, F), lambda i: (i, 0)),
            pl.BlockSpec((_BR, 1), lambda i: (i, 0)),
        ],
        out_shape=[
            jax.ShapeDtypeStruct((N, F), jnp.float32),
            jax.ShapeDtypeStruct((N, 1), jnp.float32),
        ],
    )(x, W, degp, degp)


def _tc_mid(parts, y1, dis, b1, W2):
    def body(p0_ref, p1_ref, y1_ref, dis_ref, b_ref, w_ref, y2_ref):
        dis = dis_ref[...]
        h = (p0_ref[0] + p1_ref[0] + y1_ref[...]) * dis + b_ref[...]
        hw = lax.dot_general(h, w_ref[...], (((1,), (1,)), ((), ())),
                             preferred_element_type=jnp.float32)
        y2_ref[...] = hw * dis

    return pl.pallas_call(
        body,
        grid=(_GRID,),
        in_specs=[
            pl.BlockSpec((1, _BR, F), lambda i: (0, i, 0)),
            pl.BlockSpec((1, _BR, F), lambda i: (1, i, 0)),
            pl.BlockSpec((_BR, F), lambda i: (i, 0)),
            pl.BlockSpec((_BR, 1), lambda i: (i, 0)),
            pl.BlockSpec((1, F), lambda i: (0, 0)),
            pl.BlockSpec((F, F), lambda i: (0, 0)),
        ],
        out_specs=pl.BlockSpec((_BR, F), lambda i: (i, 0)),
        out_shape=jax.ShapeDtypeStruct((N, F), jnp.float32),
    )(parts, parts, y1, dis, b1, W2)


def _tc_last(parts, y2, dis, b2):
    def body(p0_ref, p1_ref, y2_ref, dis_ref, b_ref, out_ref):
        out_ref[...] = ((p0_ref[0] + p1_ref[0] + y2_ref[...])
                        * dis_ref[...] + b_ref[...])

    return pl.pallas_call(
        body,
        grid=(_GRID,),
        in_specs=[
            pl.BlockSpec((1, _BR, F), lambda i: (0, i, 0)),
            pl.BlockSpec((1, _BR, F), lambda i: (1, i, 0)),
            pl.BlockSpec((_BR, F), lambda i: (i, 0)),
            pl.BlockSpec((_BR, 1), lambda i: (i, 0)),
            pl.BlockSpec((1, F), lambda i: (0, 0)),
        ],
        out_specs=pl.BlockSpec((_BR, F), lambda i: (i, 0)),
        out_shape=jax.ShapeDtypeStruct((N, F), jnp.float32),
    )(parts, parts, y2, dis, b2)


def kernel(x, edge_index, W1, b1, W2, b2):
    E = edge_index.shape[1]
    nchunks = NS * (C0_CNT + C1_CNT)
    rows_tot = nchunks + GRP
    e_pad = nchunks * CHUNK
    assert e_pad >= E
    e = edge_index.astype(jnp.int32)
    pad = rows_tot * CHUNK - E
    src = jnp.concatenate([e[0], jnp.zeros((pad,), jnp.int32)])
    dst = jnp.concatenate([e[1], jnp.full((pad,), N, jnp.int32)])
    edges = jnp.stack([src, dst]).reshape(2, rows_tot, CHUNK)

    zeros_blk = jnp.zeros((RPT, F), jnp.float32)
    zeros_col = jnp.zeros((RPT, DW), jnp.float32)
    ones_col = jnp.ones((CHUNK, DW), jnp.float32)
    b1r = b1.reshape(1, F)
    b2r = b2.reshape(1, F)

    degp = _sc_degree(edges, zeros_col, ones_col)
    y1, dis = _tc_first(x, W1, degp)
    s1 = _sc_aggregate(y1, edges, zeros_blk)
    y2 = _tc_mid(s1, y1, dis, b1r, W2)
    s2 = _sc_aggregate(y2, edges, zeros_blk)
    return _tc_last(s2, y2, dis, b2r)


# speedup vs baseline: 1.0256x; 1.0006x over previous
import functools

import jax
import jax.numpy as jnp
from jax import lax
from jax.experimental import pallas as pl
from jax.experimental.pallas import tpu as pltpu
from jax.experimental.pallas import tpu_sc as plsc

N = 10000
F = 128
NC, NS = 2, 16
NW = NC * NS
CHUNK = 128
RPT = 632
N_PAD = NS * RPT
DW = 128
DEG_Q = 8
GRP = 8
C0_CNT = 128
C1_CNT = 32
DEG_CNT = (C0_CNT + C1_CNT) // 2

_MESH = plsc.VectorSubcoreMesh(core_axis_name="c", subcore_axis_name="s")


def _al8(i):
    return pl.multiple_of(i, 8)


def _sc_degree(edges, zeros_col, ones_col):
    @functools.partial(
        pl.kernel,
        out_type=jax.ShapeDtypeStruct((NC, N_PAD, DW), jnp.float32),
        mesh=_MESH,
        scratch_types=[
            pltpu.VMEM((DEG_CNT, CHUNK), jnp.int32),
            pltpu.VMEM((CHUNK, DW), jnp.float32),
            pltpu.VMEM_SHARED((N_PAD, DW), jnp.float32),
            pltpu.SemaphoreType.DMA,
        ],
    )
    def k(e_hbm, z_hbm, o_hbm, out_hbm, dst2d, ones_v, acc, sem):
        c = lax.axis_index("c")
        s = lax.axis_index("s")
        wid = c * NS + s
        row0 = _al8(s * RPT)
        pltpu.sync_copy(z_hbm, acc.at[pl.ds(row0, RPT)])
        pltpu.sync_copy(o_hbm, ones_v)
        pltpu.sync_copy(e_hbm.at[1, pl.ds(_al8(wid * DEG_CNT), DEG_CNT)],
                        dst2d)
        plsc.subcore_barrier()

        def wait_one():
            pltpu.make_async_copy(ones_v, acc.at[dst2d.at[0]], sem).wait()

        def issue(j, carry):
            pltpu.async_copy(ones_v, acc.at[dst2d.at[j]], sem, add=True)

            @pl.when(j >= DEG_Q)
            def _():
                wait_one()
            return carry

        lax.fori_loop(0, DEG_CNT, issue, 0)
        for _ in range(DEG_Q):
            wait_one()
        plsc.subcore_barrier()
        pltpu.sync_copy(acc.at[pl.ds(row0, RPT)],
                        out_hbm.at[c, pl.ds(row0, RPT)])

    return k(edges, zeros_col, ones_col)


def _sc_aggregate(y, edges, zeros_blk):
    @functools.partial(
        pl.kernel,
        out_type=jax.ShapeDtypeStruct((NC, N_PAD, F), jnp.float32),
        mesh=_MESH,
        scratch_types=[
            pltpu.VMEM((2, GRP, CHUNK), jnp.int32),
            pltpu.VMEM((2, GRP, CHUNK), jnp.int32),
            pltpu.VMEM((2, CHUNK, F), jnp.float32),
            pltpu.VMEM_SHARED((N_PAD, F), jnp.float32),
            [pltpu.SemaphoreType.DMA] * 2,
            [pltpu.SemaphoreType.DMA] * 2,
            [pltpu.SemaphoreType.DMA] * 2,
            [pltpu.SemaphoreType.DMA] * 2,
        ],
    )
    def k(y_hbm, e_hbm, z_hbm, out_hbm, srcg, dstg, rows, acc,
          isems, dsems, gsems, ssems):
        c = lax.axis_index("c")
        s = lax.axis_index("s")
        cnt = jnp.where(c == 0, C0_CNT, C1_CNT)
        trip = jnp.where(c == 0, C0_CNT // (2 * GRP), C1_CNT // (2 * GRP))
        base = jnp.where(c == 0, s * C0_CNT, NS * C0_CNT + s * C1_CNT)
        row0 = _al8(s * RPT)
        pltpu.sync_copy(z_hbm, acc.at[pl.ds(row0, RPT)])

        def load_grp(gg, sl):
            at = pl.ds(_al8(base + gg * GRP), GRP)
            pltpu.async_copy(e_hbm.at[0, at], srcg.at[sl], isems[sl])
            pltpu.async_copy(e_hbm.at[1, at], dstg.at[sl], dsems[sl])

        def wait_grp(sl):
            at = pl.ds(0, GRP)
            pltpu.make_async_copy(e_hbm.at[0, at], srcg.at[sl],
                                  isems[sl]).wait()
            pltpu.make_async_copy(e_hbm.at[1, at], dstg.at[sl],
                                  dsems[sl]).wait()

        def gather(sl, kk, rb):
            pltpu.async_copy(y_hbm.at[srcg.at[sl, kk]], rows.at[rb],
                             gsems[rb])

        def wait_gather(rb):
            pltpu.make_async_copy(y_hbm.at[srcg.at[0, 0]], rows.at[rb],
                                  gsems[rb]).wait()

        def scatter(sl, kk, rb):
            pltpu.async_copy(rows.at[rb], acc.at[dstg.at[sl, kk]],
                             ssems[rb], add=True)

        def wait_scatter(rb):
            pltpu.make_async_copy(rows.at[0], acc.at[dstg.at[0, 0]],
                                  ssems[rb]).wait()

        plsc.subcore_barrier()

        def body(t, carry):
            for g2 in range(2):
                sg, other = g2, 1 - g2
                gbase = t * 2 * GRP + g2 * GRP
                gg = t * 2 + g2
                for kk in range(GRP):
                    j = gbase + kk
                    rb = kk % 2
                    rb1 = (kk + 1) % 2

                    if kk == 0:
                        @pl.when(jnp.logical_and(j >= 1, j + 1 < cnt))
                        def _():
                            wait_scatter(1)
                        @pl.when(j + GRP < cnt)
                        def _():
                            load_grp(gg + 1, other)
                    else:
                        @pl.when(jnp.logical_and(j >= 1, j + 1 < cnt))
                        def _():
                            wait_scatter(rb1)

                    if kk == GRP - 1:
                        @pl.when(j + 1 < cnt)
                        def _():
                            wait_grp(other)
                            gather(other, 0, rb1)
                    else:
                        @pl.when(j + 1 < cnt)
                        def _():
                            gather(sg, kk + 1, rb1)

                    wait_gather(rb)
                    scatter(sg, kk, rb)
            return carry

        @pl.when(c == 0)
        def _():
            load_grp(0, 0)
            wait_grp(0)
            gather(0, 0, 0)
            lax.fori_loop(0, trip, body, 0)
            wait_scatter(0)
            wait_scatter(1)

        @pl.when(c == 1)
        def _():
            def sync_body(gg, carry):
                load_grp(gg, 0)
                wait_grp(0)
                for kk in range(GRP):
                    pltpu.async_copy(y_hbm.at[srcg.at[0, kk]], rows.at[0],
                                     gsems[0]).wait()
                    pltpu.sync_copy(rows.at[0], acc.at[dstg.at[0, kk]],
                                    add=True)
                return carry

            lax.fori_loop(0, C1_CNT // GRP, sync_body, 0)

        plsc.subcore_barrier()
        pltpu.sync_copy(acc.at[pl.ds(row0, RPT)],
                        out_hbm.at[c, pl.ds(row0, RPT)])

    return k(y, edges, zeros_blk)


_BR = 2000
_GRID = (N + _BR - 1) // _BR


def _tc_first(x, W, degp):
    def body(x_ref, w_ref, d0_ref, d1_ref, y_ref, dis_ref):
        deg = d0_ref[0][:, 0:1] + d1_ref[0][:, 0:1] + 1.0
        dis = lax.rsqrt(deg)
        xw = lax.dot_general(x_ref[...], w_ref[...],
                             (((1,), (1,)), ((), ())),
                             preferred_element_type=jnp.float32)
        y_ref[...] = xw * dis
        dis_ref[...] = dis

    return pl.pallas_call(
        body,
        grid=(_GRID,),
        in_specs=[
            pl.BlockSpec((_BR, F), lambda i: (i, 0)),
            pl.BlockSpec((F, F), lambda i: (0, 0)),
            pl.BlockSpec((1, _BR, DW), lambda i: (0, i, 0)),
            pl.BlockSpec((1, _BR, DW), lambda i: (1, i, 0)),
        ],
        out_specs=[
            pl.BlockSpec((_BR, F), lambda i: (i, 0)),
            pl.BlockSpec((_BR, 1), lambda i: (i, 0)),
        ],
        out_shape=[
            jax.ShapeDtypeStruct((N, F), jnp.float32),
            jax.ShapeDtypeStruct((N, 1), jnp.float32),
        ],
    )(x, W, degp, degp)


def _tc_mid(parts, y1, dis, b1, W2):
    def body(p0_ref, p1_ref, y1_ref, dis_ref, b_ref, w_ref, y2_ref):
        dis = dis_ref[...]
        h = (p0_ref[0] + p1_ref[0] + y1_ref[...]) * dis + b_ref[...]
        hw = lax.dot_general(h, w_ref[...], (((1,), (1,)), ((), ())),
                             preferred_element_type=jnp.float32)
        y2_ref[...] = hw * dis

    return pl.pallas_call(
        body,
        grid=(_GRID,),
        in_specs=[
            pl.BlockSpec((1, _BR, F), lambda i: (0, i, 0)),
            pl.BlockSpec((1, _BR, F), lambda i: (1, i, 0)),
            pl.BlockSpec((_BR, F), lambda i: (i, 0)),
            pl.BlockSpec((_BR, 1), lambda i: (i, 0)),
            pl.BlockSpec((1, F), lambda i: (0, 0)),
            pl.BlockSpec((F, F), lambda i: (0, 0)),
        ],
        out_specs=pl.BlockSpec((_BR, F), lambda i: (i, 0)),
        out_shape=jax.ShapeDtypeStruct((N, F), jnp.float32),
    )(parts, parts, y1, dis, b1, W2)


def _tc_last(parts, y2, dis, b2):
    def body(p0_ref, p1_ref, y2_ref, dis_ref, b_ref, out_ref):
        out_ref[...] = ((p0_ref[0] + p1_ref[0] + y2_ref[...])
                        * dis_ref[...] + b_ref[...])

    return pl.pallas_call(
        body,
        grid=(_GRID,),
        in_specs=[
            pl.BlockSpec((1, _BR, F), lambda i: (0, i, 0)),
            pl.BlockSpec((1, _BR, F), lambda i: (1, i, 0)),
            pl.BlockSpec((_BR, F), lambda i: (i, 0)),
            pl.BlockSpec((_BR, 1), lambda i: (i, 0)),
            pl.BlockSpec((1, F), lambda i: (0, 0)),
        ],
        out_specs=pl.BlockSpec((_BR, F), lambda i: (i, 0)),
        out_shape=jax.ShapeDtypeStruct((N, F), jnp.float32),
    )(parts, parts, y2, dis, b2)


def kernel(x, edge_index, W1, b1, W2, b2):
    E = edge_index.shape[1]
    nchunks = NS * (C0_CNT + C1_CNT)
    rows_tot = nchunks + GRP
    e_pad = nchunks * CHUNK
    assert e_pad >= E
    e = edge_index.astype(jnp.int32)
    pad = rows_tot * CHUNK - E
    src = jnp.concatenate([e[0], jnp.zeros((pad,), jnp.int32)])
    dst = jnp.concatenate([e[1], jnp.full((pad,), N, jnp.int32)])
    edges = jnp.stack([src, dst]).reshape(2, rows_tot, CHUNK)

    zeros_blk = jnp.zeros((RPT, F), jnp.float32)
    zeros_col = jnp.zeros((RPT, DW), jnp.float32)
    ones_col = jnp.ones((CHUNK, DW), jnp.float32)
    b1r = b1.reshape(1, F)
    b2r = b2.reshape(1, F)

    degp = _sc_degree(edges, zeros_col, ones_col)
    y1, dis = _tc_first(x, W1, degp)
    s1 = _sc_aggregate(y1, edges, zeros_blk)
    y2 = _tc_mid(s1, y1, dis, b1r, W2)
    s2 = _sc_aggregate(y2, edges, zeros_blk)
    return _tc_last(s2, y2, dis, b2r)


# speedup vs baseline: 1.0964x; 1.0690x over previous
import functools

import jax
import jax.numpy as jnp
from jax import lax
from jax.experimental import pallas as pl
from jax.experimental.pallas import tpu as pltpu
from jax.experimental.pallas import tpu_sc as plsc

N = 10000
F = 128
NC, NS = 2, 16
NW = NC * NS
CHUNK = 128
RPT = 632
N_PAD = NS * RPT
DW = 128
DEG_Q = 8
GRP = 8
C0_CNT = 128
C1_CNT = 32
DEG_CNT = (C0_CNT + C1_CNT) // 2

_MESH = plsc.VectorSubcoreMesh(core_axis_name="c", subcore_axis_name="s")


def _al8(i):
    return pl.multiple_of(i, 8)


def _sc_degree(edges, zeros_col, ones_col):
    @functools.partial(
        pl.kernel,
        out_type=jax.ShapeDtypeStruct((NC, N_PAD, DW), jnp.float32),
        mesh=_MESH,
        scratch_types=[
            pltpu.VMEM((DEG_CNT, CHUNK), jnp.int32),
            pltpu.VMEM((CHUNK, DW), jnp.float32),
            pltpu.VMEM_SHARED((N_PAD, DW), jnp.float32),
            pltpu.SemaphoreType.DMA,
        ],
    )
    def k(e_hbm, z_hbm, o_hbm, out_hbm, dst2d, ones_v, acc, sem):
        c = lax.axis_index("c")
        s = lax.axis_index("s")
        wid = c * NS + s
        row0 = _al8(s * RPT)
        pltpu.sync_copy(z_hbm, acc.at[pl.ds(row0, RPT)])
        pltpu.sync_copy(o_hbm, ones_v)
        pltpu.sync_copy(e_hbm.at[1, pl.ds(_al8(wid * DEG_CNT), DEG_CNT)],
                        dst2d)
        plsc.subcore_barrier()

        def wait_one():
            pltpu.make_async_copy(ones_v, acc.at[dst2d.at[0]], sem).wait()

        def issue(j, carry):
            pltpu.async_copy(ones_v, acc.at[dst2d.at[j]], sem, add=True)

            @pl.when(j >= DEG_Q)
            def _():
                wait_one()
            return carry

        lax.fori_loop(0, DEG_CNT, issue, 0)
        for _ in range(DEG_Q):
            wait_one()
        plsc.subcore_barrier()
        pltpu.sync_copy(acc.at[pl.ds(row0, RPT)],
                        out_hbm.at[c, pl.ds(row0, RPT)])

    return k(edges, zeros_col, ones_col)


def _sc_aggregate(y, edges, edges2d, zeros_blk):
    @functools.partial(
        pl.kernel,
        out_type=jax.ShapeDtypeStruct((NC, N_PAD, F), jnp.float32),
        mesh=_MESH,
        scratch_types=[
            pltpu.VMEM((2, GRP, CHUNK), jnp.int32),
            pltpu.VMEM((2, GRP, CHUNK), jnp.int32),
            pltpu.VMEM((CHUNK,), jnp.int32),
            pltpu.VMEM((CHUNK,), jnp.int32),
            pltpu.VMEM((2, CHUNK, F), jnp.float32),
            pltpu.VMEM_SHARED((N_PAD, F), jnp.float32),
            [pltpu.SemaphoreType.DMA] * 2,
            [pltpu.SemaphoreType.DMA] * 2,
            [pltpu.SemaphoreType.DMA] * 2,
            [pltpu.SemaphoreType.DMA] * 2,
        ],
    )
    def k(y_hbm, e_hbm, e2_hbm, z_hbm, out_hbm, srcg, dstg, src_idx, dst_idx,
          rows, acc, isems, dsems, gsems, ssems):
        c = lax.axis_index("c")
        s = lax.axis_index("s")
        cnt = jnp.where(c == 0, C0_CNT, C1_CNT)
        trip = jnp.where(c == 0, C0_CNT // (2 * GRP), C1_CNT // (2 * GRP))
        base = jnp.where(c == 0, s * C0_CNT, NS * C0_CNT + s * C1_CNT)
        row0 = _al8(s * RPT)
        pltpu.sync_copy(z_hbm, acc.at[pl.ds(row0, RPT)])

        def load_grp(gg, sl):
            at = pl.ds(_al8(base + gg * GRP), GRP)
            pltpu.async_copy(e_hbm.at[0, at], srcg.at[sl], isems[sl])
            pltpu.async_copy(e_hbm.at[1, at], dstg.at[sl], dsems[sl])

        def wait_grp(sl):
            at = pl.ds(0, GRP)
            pltpu.make_async_copy(e_hbm.at[0, at], srcg.at[sl],
                                  isems[sl]).wait()
            pltpu.make_async_copy(e_hbm.at[1, at], dstg.at[sl],
                                  dsems[sl]).wait()

        def gather(sl, kk, rb):
            pltpu.async_copy(y_hbm.at[srcg.at[sl, kk]], rows.at[rb],
                             gsems[rb])

        def wait_gather(rb):
            pltpu.make_async_copy(y_hbm.at[srcg.at[0, 0]], rows.at[rb],
                                  gsems[rb]).wait()

        def scatter(sl, kk, rb):
            pltpu.async_copy(rows.at[rb], acc.at[dstg.at[sl, kk]],
                             ssems[rb], add=True)

        def wait_scatter(rb):
            pltpu.make_async_copy(rows.at[0], acc.at[dstg.at[0, 0]],
                                  ssems[rb]).wait()

        plsc.subcore_barrier()

        def body(t, carry):
            for g2 in range(2):
                sg, other = g2, 1 - g2
                gbase = t * 2 * GRP + g2 * GRP
                gg = t * 2 + g2
                for kk in range(GRP):
                    j = gbase + kk
                    rb = kk % 2
                    rb1 = (kk + 1) % 2

                    if kk == 0:
                        @pl.when(jnp.logical_and(j >= 1, j + 1 < cnt))
                        def _():
                            wait_scatter(1)
                        @pl.when(j + GRP < cnt)
                        def _():
                            load_grp(gg + 1, other)
                    else:
                        @pl.when(jnp.logical_and(j >= 1, j + 1 < cnt))
                        def _():
                            wait_scatter(rb1)

                    if kk == GRP - 1:
                        @pl.when(j + 1 < cnt)
                        def _():
                            wait_grp(other)
                            gather(other, 0, rb1)
                    else:
                        @pl.when(j + 1 < cnt)
                        def _():
                            gather(sg, kk + 1, rb1)

                    wait_gather(rb)
                    scatter(sg, kk, rb)
            return carry

        @pl.when(c == 0)
        def _():
            load_grp(0, 0)
            wait_grp(0)
            gather(0, 0, 0)
            lax.fori_loop(0, trip, body, 0)
            wait_scatter(0)
            wait_scatter(1)

        @pl.when(c == 1)
        def _():
            e0 = (NS * C0_CNT + s * C1_CNT) * CHUNK

            def sync_body(j, carry):
                eb = e0 + j * CHUNK
                pltpu.sync_copy(e2_hbm.at[0, pl.ds(eb, CHUNK)], src_idx)
                pltpu.sync_copy(e2_hbm.at[1, pl.ds(eb, CHUNK)], dst_idx)
                pltpu.async_copy(y_hbm.at[src_idx], rows.at[0],
                                 gsems[0]).wait()
                pltpu.sync_copy(rows.at[0], acc.at[dst_idx], add=True)
                return carry

            lax.fori_loop(0, C1_CNT, sync_body, 0)

        plsc.subcore_barrier()
        pltpu.sync_copy(acc.at[pl.ds(row0, RPT)],
                        out_hbm.at[c, pl.ds(row0, RPT)])

    return k(y, edges, edges2d, zeros_blk)


_BR = 2000
_GRID = (N + _BR - 1) // _BR


def _tc_first(x, W, degp):
    def body(x_ref, w_ref, d0_ref, d1_ref, y_ref, dis_ref):
        deg = d0_ref[0][:, 0:1] + d1_ref[0][:, 0:1] + 1.0
        dis = lax.rsqrt(deg)
        xw = lax.dot_general(x_ref[...], w_ref[...],
                             (((1,), (1,)), ((), ())),
                             preferred_element_type=jnp.float32)
        y_ref[...] = xw * dis
        dis_ref[...] = dis

    return pl.pallas_call(
        body,
        grid=(_GRID,),
        in_specs=[
            pl.BlockSpec((_BR, F), lambda i: (i, 0)),
            pl.BlockSpec((F, F), lambda i: (0, 0)),
            pl.BlockSpec((1, _BR, DW), lambda i: (0, i, 0)),
            pl.BlockSpec((1, _BR, DW), lambda i: (1, i, 0)),
        ],
        out_specs=[
            pl.BlockSpec((_BR, F), lambda i: (i, 0)),
            pl.BlockSpec((_BR, 1), lambda i: (i, 0)),
        ],
        out_shape=[
            jax.ShapeDtypeStruct((N, F), jnp.float32),
            jax.ShapeDtypeStruct((N, 1), jnp.float32),
        ],
    )(x, W, degp, degp)


def _tc_mid(parts, y1, dis, b1, W2):
    def body(p0_ref, p1_ref, y1_ref, dis_ref, b_ref, w_ref, y2_ref):
        dis = dis_ref[...]
        h = (p0_ref[0] + p1_ref[0] + y1_ref[...]) * dis + b_ref[...]
        hw = lax.dot_general(h, w_ref[...], (((1,), (1,)), ((), ())),
                             preferred_element_type=jnp.float32)
        y2_ref[...] = hw * dis

    return pl.pallas_call(
        body,
        grid=(_GRID,),
        in_specs=[
            pl.BlockSpec((1, _BR, F), lambda i: (0, i, 0)),
            pl.BlockSpec((1, _BR, F), lambda i: (1, i, 0)),
            pl.BlockSpec((_BR, F), lambda i: (i, 0)),
            pl.BlockSpec((_BR, 1), lambda i: (i, 0)),
            pl.BlockSpec((1, F), lambda i: (0, 0)),
            pl.BlockSpec((F, F), lambda i: (0, 0)),
        ],
        out_specs=pl.BlockSpec((_BR, F), lambda i: (i, 0)),
        out_shape=jax.ShapeDtypeStruct((N, F), jnp.float32),
    )(parts, parts, y1, dis, b1, W2)


def _tc_last(parts, y2, dis, b2):
    def body(p0_ref, p1_ref, y2_ref, dis_ref, b_ref, out_ref):
        out_ref[...] = ((p0_ref[0] + p1_ref[0] + y2_ref[...])
                        * dis_ref[...] + b_ref[...])

    return pl.pallas_call(
        body,
        grid=(_GRID,),
        in_specs=[
            pl.BlockSpec((1, _BR, F), lambda i: (0, i, 0)),
            pl.BlockSpec((1, _BR, F), lambda i: (1, i, 0)),
            pl.BlockSpec((_BR, F), lambda i: (i, 0)),
            pl.BlockSpec((_BR, 1), lambda i: (i, 0)),
            pl.BlockSpec((1, F), lambda i: (0, 0)),
        ],
        out_specs=pl.BlockSpec((_BR, F), lambda i: (i, 0)),
        out_shape=jax.ShapeDtypeStruct((N, F), jnp.float32),
    )(parts, parts, y2, dis, b2)


def kernel(x, edge_index, W1, b1, W2, b2):
    E = edge_index.shape[1]
    nchunks = NS * (C0_CNT + C1_CNT)
    rows_tot = nchunks + GRP
    e_pad = nchunks * CHUNK
    assert e_pad >= E
    e = edge_index.astype(jnp.int32)
    pad = rows_tot * CHUNK - E
    src = jnp.concatenate([e[0], jnp.zeros((pad,), jnp.int32)])
    dst = jnp.concatenate([e[1], jnp.full((pad,), N, jnp.int32)])
    edges = jnp.stack([src, dst]).reshape(2, rows_tot, CHUNK)
    edges2d = edges.reshape(2, rows_tot * CHUNK)

    zeros_blk = jnp.zeros((RPT, F), jnp.float32)
    zeros_col = jnp.zeros((RPT, DW), jnp.float32)
    ones_col = jnp.ones((CHUNK, DW), jnp.float32)
    b1r = b1.reshape(1, F)
    b2r = b2.reshape(1, F)

    degp = _sc_degree(edges, zeros_col, ones_col)
    y1, dis = _tc_first(x, W1, degp)
    s1 = _sc_aggregate(y1, edges, edges2d, zeros_blk)
    y2 = _tc_mid(s1, y1, dis, b1r, W2)
    s2 = _sc_aggregate(y2, edges, edges2d, zeros_blk)
    return _tc_last(s2, y2, dis, b2r)
